# Initial kernel scaffold; baseline (speedup 1.0000x reference)
import functools

import jax
import jax.numpy as jnp
from jax import lax
from jax.experimental import pallas as pl
from jax.experimental.pallas import tpu as pltpu
from jax.experimental.pallas import tpu_sc as plsc

_EPS = 1e-08
_NUM_CORES = 2
_NUM_SUBCORES = 16
_NW = _NUM_CORES * _NUM_SUBCORES
_W = 8


def _rsqrt(s):
    i = lax.bitcast_convert_type(s, jnp.int32)
    i = jnp.int32(0x5F3759DF) - lax.shift_right_logical(i, 1)
    y = lax.bitcast_convert_type(i, jnp.float32)
    for _ in range(3):
        y = y * (jnp.float32(1.5) - jnp.float32(0.5) * s * y * y)
    return y


@functools.cache
def _build_kernel(nb_pad):
    chunk = nb_pad // _NW
    groups = chunk // 16

    mesh = plsc.VectorSubcoreMesh(
        core_axis_name="c",
        subcore_axis_name="s",
        num_cores=_NUM_CORES,
        num_subcores=_NUM_SUBCORES,
    )

    @functools.partial(
        pl.kernel,
        out_type=jax.ShapeDtypeStruct((_NW, 16), jnp.float32),
        mesh=mesh,
        scratch_types=[
            pltpu.VMEM((chunk,), jnp.int32),
            pltpu.VMEM((chunk,), jnp.int32),
            pltpu.VMEM((chunk, _W), jnp.float32),
            pltpu.VMEM((chunk, _W), jnp.float32),
            pltpu.VMEM((16,), jnp.float32),
            pltpu.SemaphoreType.DMA,
        ],
    )
    def bond_loss(tab_hbm, s_hbm, e_hbm, out_hbm,
                  idx_s, idx_e, rows_s, rows_e, acc_v, sem):
        wid = lax.axis_index("s") * _NUM_CORES + lax.axis_index("c")
        base = wid * chunk
        pltpu.sync_copy(s_hbm.at[pl.ds(base, chunk)], idx_s)
        pltpu.sync_copy(e_hbm.at[pl.ds(base, chunk)], idx_e)
        cp_s = pltpu.async_copy(tab_hbm.at[idx_s], rows_s, sem)
        cp_e = pltpu.async_copy(tab_hbm.at[idx_e], rows_e, sem)
        cp_s.wait()
        cp_e.wait()

        lanes = jax.lax.iota(jnp.int32, 16)

        def body(g, acc):
            r = g * 16 + lanes

            def col(rows, c):
                return plsc.load_gather(
                    rows, [r, jnp.full((16,), c, jnp.int32)])

            dx = col(rows_s, 0) - col(rows_e, 0)
            dy = col(rows_s, 1) - col(rows_e, 1)
            dz = col(rows_s, 2) - col(rows_e, 2)
            gx = col(rows_s, 4) - col(rows_e, 4)
            gy = col(rows_s, 5) - col(rows_e, 5)
            gz = col(rows_s, 6) - col(rows_e, 6)
            sp = dx * dx + dy * dy + dz * dz + jnp.float32(_EPS)
            sg = gx * gx + gy * gy + gz * gz + jnp.float32(_EPS)
            lp = sp * _rsqrt(sp)
            lg = sg * _rsqrt(sg)
            d = lp - lg
            return acc + d * d

        acc = lax.fori_loop(0, groups, body, jnp.zeros((16,), jnp.float32))
        acc_v[...] = acc
        pltpu.sync_copy(acc_v, out_hbm.at[wid])

    return bond_loss


@jax.jit
def kernel(x_pred, x_gt, bonds):
    n_atoms = x_pred.shape[0]
    nb = bonds.shape[0]
    nb_pad = ((nb + _NW * 8 - 1) // (_NW * 8)) * (_NW * 8)
    pad = nb_pad - nb

    z = jnp.zeros((n_atoms, 1), jnp.float32)
    tab = jnp.concatenate(
        [x_pred.astype(jnp.float32), z, x_gt.astype(jnp.float32), z], axis=1)
    b32 = bonds.astype(jnp.int32)
    zpad = jnp.zeros((pad,), jnp.int32)
    starts = jnp.concatenate([b32[:, 0], zpad])
    ends = jnp.concatenate([b32[:, 1], zpad])

    parts = _build_kernel(nb_pad)(tab, starts, ends)
    return jnp.sum(parts) / jnp.float32(nb)


# speedup vs baseline: 15.1734x; 15.1734x over previous
import functools

import jax
import jax.numpy as jnp
from jax import lax
from jax.experimental import pallas as pl
from jax.experimental.pallas import tpu as pltpu
from jax.experimental.pallas import tpu_sc as plsc

_EPS = 1e-08
_NUM_CORES = 2
_NUM_SUBCORES = 16
_NW = _NUM_CORES * _NUM_SUBCORES
_W = 8


def _rsqrt(s):
    i = lax.bitcast_convert_type(s, jnp.int32)
    i = jnp.int32(0x5F3759DF) - lax.shift_right_logical(i, 1)
    y = lax.bitcast_convert_type(i, jnp.float32)
    for _ in range(3):
        y = y * (jnp.float32(1.5) - jnp.float32(0.5) * s * y * y)
    return y


@functools.cache
def _build_kernel(nb_pad):
    chunk = nb_pad // _NW
    groups = chunk // 16

    mesh = plsc.VectorSubcoreMesh(
        core_axis_name="c",
        subcore_axis_name="s",
        num_cores=_NUM_CORES,
        num_subcores=_NUM_SUBCORES,
    )

    @functools.partial(
        pl.kernel,
        out_type=jax.ShapeDtypeStruct((_NW, 16), jnp.float32),
        mesh=mesh,
        compiler_params=pltpu.CompilerParams(
            needs_layout_passes=False, use_tc_tiling_on_sc=False),
        scratch_types=[
            pltpu.VMEM((chunk,), jnp.int32),
            pltpu.VMEM((chunk,), jnp.int32),
            pltpu.VMEM((chunk, _W), jnp.float32),
            pltpu.VMEM((chunk, _W), jnp.float32),
            pltpu.VMEM((16,), jnp.float32),
            pltpu.SemaphoreType.DMA,
        ],
    )
    def bond_loss(tab_hbm, s_hbm, e_hbm, out_hbm,
                  idx_s, idx_e, rows_s, rows_e, acc_v, sem):
        wid = lax.axis_index("s") * _NUM_CORES + lax.axis_index("c")
        base = wid * chunk
        pltpu.sync_copy(s_hbm.at[pl.ds(base, chunk)], idx_s)
        pltpu.sync_copy(e_hbm.at[pl.ds(base, chunk)], idx_e)
        cp_s = pltpu.async_copy(tab_hbm.at[idx_s], rows_s, sem)
        cp_e = pltpu.async_copy(tab_hbm.at[idx_e], rows_e, sem)
        cp_s.wait()
        cp_e.wait()

        lanes = jax.lax.iota(jnp.int32, 16)

        def body(g, acc):
            r = g * 16 + lanes

            def col(rows, c):
                return plsc.load_gather(
                    rows, [r, jnp.full((16,), c, jnp.int32)])

            dx = col(rows_s, 0) - col(rows_e, 0)
            dy = col(rows_s, 1) - col(rows_e, 1)
            dz = col(rows_s, 2) - col(rows_e, 2)
            gx = col(rows_s, 4) - col(rows_e, 4)
            gy = col(rows_s, 5) - col(rows_e, 5)
            gz = col(rows_s, 6) - col(rows_e, 6)
            sp = dx * dx + dy * dy + dz * dz + jnp.float32(_EPS)
            sg = gx * gx + gy * gy + gz * gz + jnp.float32(_EPS)
            lp = sp * _rsqrt(sp)
            lg = sg * _rsqrt(sg)
            d = lp - lg
            return acc + d * d

        acc = lax.fori_loop(0, groups, body, jnp.zeros((16,), jnp.float32))
        acc_v[...] = acc
        pltpu.sync_copy(acc_v, out_hbm.at[wid])

    return bond_loss


@jax.jit
def kernel(x_pred, x_gt, bonds):
    n_atoms = x_pred.shape[0]
    nb = bonds.shape[0]
    nb_pad = ((nb + _NW * 8 - 1) // (_NW * 8)) * (_NW * 8)
    pad = nb_pad - nb

    z = jnp.zeros((n_atoms, 1), jnp.float32)
    tab = jnp.concatenate(
        [x_pred.astype(jnp.float32), z, x_gt.astype(jnp.float32), z], axis=1)
    b32 = bonds.astype(jnp.int32)
    zpad = jnp.zeros((pad,), jnp.int32)
    starts = jnp.concatenate([b32[:, 0], zpad])
    ends = jnp.concatenate([b32[:, 1], zpad])

    parts = _build_kernel(nb_pad)(tab, starts, ends)
    return jnp.sum(parts) / jnp.float32(nb)
